# Initial kernel scaffold; baseline (speedup 1.0000x reference)
import jax
import jax.numpy as jnp
from jax.experimental import pallas as pl


def kernel(x, edge_attr, params, Wf, bf, edge_index, batch):
    raise NotImplementedError("write your pallas kernel here")


# speedup vs baseline: 2.5255x; 2.5255x over previous
import functools

import jax
import jax.numpy as jnp
from jax import lax
from jax.experimental import pallas as pl
from jax.experimental.pallas import tpu as pltpu
from jax.experimental.pallas import tpu_sc as plsc

N = 10000
E = 320000
DE = 16
HD = 128
G = 64
C = 10

NC = 2
NS = 16
NW = NC * NS

EPT = E // NW
BLK = 80
NB = EPT // BLK
ZCH = 80
NCH = N // ZCH
KMAX = (NCH + NS - 1) // NS

_HIGH = jax.lax.Precision.HIGHEST


def _sc_body(h_hbm, ea_hbm, src_hbm, dst_hbm, out_hbm,
             src_v, dst_v, hrow_v, ea_v, msg_v, agg_sh, sem):
    c = lax.axis_index("c")
    s = lax.axis_index("s")
    wid = s * NC + c

    def zfill(e, _):
        for cc in range(HD // 16):
            msg_v[e, pl.ds(cc * 16, 16)] = jnp.zeros((16,), jnp.float32)
        return 0
    lax.fori_loop(0, BLK, zfill, 0)
    for k in range(KMAX):
        ch = s + k * NS
        @pl.when(ch < NCH)
        def _():
            pltpu.sync_copy(msg_v, agg_sh.at[pl.ds(ch * ZCH, ZCH), :])
    plsc.subcore_barrier()

    def block(b, _):
        base = wid * EPT + b * BLK
        pltpu.sync_copy(src_hbm.at[pl.ds(base, BLK)], src_v)
        pltpu.sync_copy(dst_hbm.at[pl.ds(base, BLK)], dst_v)
        cp = pltpu.async_copy(h_hbm.at[src_v], hrow_v, sem)
        pltpu.sync_copy(ea_hbm.at[pl.ds(base, BLK), :], ea_v)
        cp.wait()

        def row(e, _):
            for cc in range(HD // 16):
                sl = pl.ds(cc * 16, 16)
                msg_v[e, sl] = jnp.maximum(hrow_v[e, sl] + ea_v[e, sl], 0.0)
            return 0
        lax.fori_loop(0, BLK, row, 0)

        pltpu.sync_copy(msg_v, agg_sh.at[dst_v], add=True)
        return 0
    lax.fori_loop(0, NB, block, 0)

    plsc.subcore_barrier()

    for k in range(KMAX):
        ch = s + k * NS
        @pl.when(ch < NCH)
        def _():
            pltpu.sync_copy(agg_sh.at[pl.ds(ch * ZCH, ZCH), :],
                            out_hbm.at[pl.ds(c * N + ch * ZCH, ZCH), :])


_sc_msg = functools.partial(
    pl.kernel,
    out_type=jax.ShapeDtypeStruct((2 * N, HD), jnp.float32),
    mesh=plsc.VectorSubcoreMesh(core_axis_name="c", subcore_axis_name="s"),
    scratch_types=[
        pltpu.VMEM((BLK,), jnp.int32),
        pltpu.VMEM((BLK,), jnp.int32),
        pltpu.VMEM((BLK, HD), jnp.float32),
        pltpu.VMEM((BLK, HD), jnp.float32),
        pltpu.VMEM((BLK, HD), jnp.float32),
        pltpu.VMEM_SHARED((N, HD), jnp.float32),
        pltpu.SemaphoreType.DMA,
    ],
)(_sc_body)


_BE = 4000


def _embed_body(ea_ref, we_ref, be_ref, o0_ref, o1_ref, o2_ref):
    ea = ea_ref[...]
    outs = (o0_ref, o1_ref, o2_ref)
    for l in range(3):
        o = jnp.dot(ea, we_ref[l], preferred_element_type=jnp.float32,
                    precision=_HIGH)
        outs[l][...] = o + be_ref[l]


def _edge_embed(edge_attr, we_stack, be_stack):
    nblk = E // _BE
    return pl.pallas_call(
        _embed_body,
        grid=(nblk,),
        in_specs=[
            pl.BlockSpec((_BE, DE), lambda i: (i, 0)),
            pl.BlockSpec((3, DE, HD), lambda i: (0, 0, 0)),
            pl.BlockSpec((3, 1, HD), lambda i: (0, 0, 0)),
        ],
        out_specs=[
            pl.BlockSpec((_BE, HD), lambda i: (i, 0)),
            pl.BlockSpec((_BE, HD), lambda i: (i, 0)),
            pl.BlockSpec((_BE, HD), lambda i: (i, 0)),
        ],
        out_shape=[jax.ShapeDtypeStruct((E, HD), jnp.float32)] * 3,
    )(edge_attr, we_stack, be_stack)


def _bn(y, g, b):
    m = jnp.sum(y, axis=0, keepdims=True) * (1.0 / N)
    d = y - m
    v = jnp.sum(d * d, axis=0, keepdims=True) * (1.0 / N)
    return d * jax.lax.rsqrt(v + 1e-5) * g + b


def _lrelu(y):
    return jnp.where(y >= 0, y, 0.01 * y)


def _mlp_body(h_ref, agg_ref, eps_ref, w1_ref, b1_ref, g1_ref, bb1_ref,
              w2_ref, b2_ref, g2_ref, bb2_ref, out_ref):
    h = h_ref[...]
    z = (1.0 + eps_ref[0, 0]) * h + agg_ref[:N, :] + agg_ref[N:, :]
    y = jnp.dot(z, w1_ref[...], preferred_element_type=jnp.float32,
                precision=_HIGH) + b1_ref[...]
    y = _lrelu(_bn(y, g1_ref[...], bb1_ref[...]))
    y = jnp.dot(y, w2_ref[...], preferred_element_type=jnp.float32,
                precision=_HIGH) + b2_ref[...]
    out_ref[...] = _lrelu(_bn(y, g2_ref[...], bb2_ref[...]))


def _layer_mlp(h, aggp, p):
    args = (h, aggp, p["eps"].reshape(1, 1),
            p["W1"], p["b1"].reshape(1, HD), p["g1"].reshape(1, HD),
            p["bb1"].reshape(1, HD),
            p["W2"], p["b2"].reshape(1, HD), p["g2"].reshape(1, HD),
            p["bb2"].reshape(1, HD))
    return pl.pallas_call(
        _mlp_body,
        out_shape=jax.ShapeDtypeStruct((N, HD), jnp.float32),
    )(*args)


def _final_body(h_ref, batch_ref, wft_ref, wfb_ref, bf_ref, out_ref):
    h = h_ref[...]
    b = batch_ref[...]
    gi = jax.lax.broadcasted_iota(jnp.int32, (N, G), 1)
    oh = (b == gi).astype(jnp.float32)
    pool = jax.lax.dot_general(oh, h, (((0,), (0,)), ((), ())),
                               preferred_element_type=jnp.float32,
                               precision=_HIGH)
    pe = jnp.dot(oh, pool, preferred_element_type=jnp.float32,
                 precision=_HIGH)
    logits = (jnp.dot(h, wft_ref[...], preferred_element_type=jnp.float32,
                      precision=_HIGH)
              + jnp.dot(pe, wfb_ref[...], preferred_element_type=jnp.float32,
                        precision=_HIGH)
              + bf_ref[...])
    m = jnp.max(logits, axis=1, keepdims=True)
    ex = jnp.exp(logits - m)
    out_ref[...] = ex / jnp.sum(ex, axis=1, keepdims=True)


def _final(h, batch2d, wf_top, wf_bot, bf):
    return pl.pallas_call(
        _final_body,
        out_shape=jax.ShapeDtypeStruct((N, C), jnp.float32),
    )(h, batch2d, wf_top, wf_bot, bf.reshape(1, C))


def kernel(x, edge_attr, params, Wf, bf, edge_index, batch):
    src = edge_index[0].astype(jnp.int32)
    dst = edge_index[1].astype(jnp.int32)
    we_stack = jnp.stack([p["We"] for p in params])
    be_stack = jnp.stack([p["be"].reshape(1, HD) for p in params])
    eas = _edge_embed(edge_attr, we_stack, be_stack)

    h = x
    for l, p in enumerate(params):
        aggp = _sc_msg(h, eas[l], src, dst)
        h = _layer_mlp(h, aggp, p)

    return _final(h, batch.astype(jnp.int32).reshape(N, 1),
                  Wf[:HD], Wf[HD:], bf)


# speedup vs baseline: 3.8786x; 1.5358x over previous
import functools

import jax
import jax.numpy as jnp
from jax import lax
from jax.experimental import pallas as pl
from jax.experimental.pallas import tpu as pltpu
from jax.experimental.pallas import tpu_sc as plsc

N = 10000
E = 320000
DE = 16
HD = 128
G = 64
C = 10

NC = 2
NS = 16
NW = NC * NS

EPT = E // NW
BLK = 40
NB = EPT // BLK
SCB = 50
NSC = NB // SCB
ZCH = 40
NCH = N // ZCH
KMAX = (NCH + NS - 1) // NS

_HIGH = jax.lax.Precision.HIGHEST


def _sc_body(h_hbm, ea_hbm, src_hbm, dst4_hbm, out_hbm,
             sidx_v, didx_v, hrow_v, ea_v, msg_v, agg_sh,
             gsem, esem, ssem):
    c = lax.axis_index("c")
    s = lax.axis_index("s")
    wid = s * NC + c

    def zfill(e, _):
        for cc in range(HD // 16):
            msg_v[0, e, pl.ds(cc * 16, 16)] = jnp.zeros((16,), jnp.float32)
        return 0
    lax.fori_loop(0, BLK, zfill, 0)
    for k in range(KMAX):
        ch = s + k * NS
        @pl.when(ch < NCH)
        def _():
            pltpu.sync_copy(msg_v.at[0], agg_sh.at[pl.ds(ch * ZCH, ZCH), :])
    plsc.subcore_barrier()

    def issue(sc, bl, par):
        base = wid * EPT + sc * SCB * BLK + bl * BLK
        pltpu.async_copy(h_hbm.at[sidx_v.at[pl.ds(bl * BLK, BLK)]],
                         hrow_v.at[par], gsem.at[par])
        pltpu.async_copy(ea_hbm.at[pl.ds(base, BLK), :], ea_v.at[par],
                         esem.at[par])

    def wait_in(par):
        pltpu.make_async_copy(h_hbm.at[pl.ds(0, BLK), :], hrow_v.at[par],
                              gsem.at[par]).wait()
        pltpu.make_async_copy(ea_hbm.at[pl.ds(0, BLK), :], ea_v.at[par],
                              esem.at[par]).wait()

    def drain_scatter(par):
        pltpu.make_async_copy(ea_hbm.at[pl.ds(0, BLK), :], msg_v.at[par],
                              ssem.at[par]).wait()

    def chunk(sc, _):
        pltpu.sync_copy(src_hbm.at[pl.ds(wid * EPT + sc * SCB * BLK,
                                         SCB * BLK)], sidx_v)
        pltpu.sync_copy(dst4_hbm.at[wid, sc], didx_v)
        issue(sc, 0, 0)

        def pair(i, _):
            bl0 = i * 2
            for par in range(2):
                bl = bl0 + par
                nxt = bl + 1
                @pl.when(nxt < SCB)
                def _():
                    issue(sc, nxt, 1 - par)
                wait_in(par)
                @pl.when(bl >= 2)
                def _():
                    drain_scatter(par)

                def row(e, _):
                    for cc in range(HD // 16):
                        sl = pl.ds(cc * 16, 16)
                        msg_v[par, e, sl] = jnp.maximum(
                            hrow_v[par, e, sl] + ea_v[par, e, sl], 0.0)
                    return 0
                lax.fori_loop(0, BLK, row, 0)

                pltpu.async_copy(msg_v.at[par], agg_sh.at[didx_v.at[bl]],
                                 ssem.at[par], add=True)
            return 0
        lax.fori_loop(0, SCB // 2, pair, 0)
        drain_scatter(0)
        drain_scatter(1)
        return 0
    lax.fori_loop(0, NSC, chunk, 0)

    plsc.subcore_barrier()

    for k in range(KMAX):
        ch = s + k * NS
        @pl.when(ch < NCH)
        def _():
            pltpu.sync_copy(agg_sh.at[pl.ds(ch * ZCH, ZCH), :],
                            out_hbm.at[pl.ds(c * N + ch * ZCH, ZCH), :])


_sc_msg = functools.partial(
    pl.kernel,
    out_type=jax.ShapeDtypeStruct((2 * N, HD), jnp.float32),
    mesh=plsc.VectorSubcoreMesh(core_axis_name="c", subcore_axis_name="s"),
    scratch_types=[
        pltpu.VMEM((SCB * BLK,), jnp.int32),
        pltpu.VMEM((SCB, BLK), jnp.int32),
        pltpu.VMEM((2, BLK, HD), jnp.float32),
        pltpu.VMEM((2, BLK, HD), jnp.float32),
        pltpu.VMEM((2, BLK, HD), jnp.float32),
        pltpu.VMEM_SHARED((N, HD), jnp.float32),
        pltpu.SemaphoreType.DMA((2,)),
        pltpu.SemaphoreType.DMA((2,)),
        pltpu.SemaphoreType.DMA((2,)),
    ],
)(_sc_body)


_BE = 4000


def _embed_body(ea_ref, we_ref, be_ref, o0_ref, o1_ref, o2_ref):
    ea = ea_ref[...]
    outs = (o0_ref, o1_ref, o2_ref)
    for l in range(3):
        o = jnp.dot(ea, we_ref[l], preferred_element_type=jnp.float32,
                    precision=_HIGH)
        outs[l][...] = o + be_ref[l]


def _edge_embed(edge_attr, we_stack, be_stack):
    nblk = E // _BE
    return pl.pallas_call(
        _embed_body,
        grid=(nblk,),
        in_specs=[
            pl.BlockSpec((_BE, DE), lambda i: (i, 0)),
            pl.BlockSpec((3, DE, HD), lambda i: (0, 0, 0)),
            pl.BlockSpec((3, 1, HD), lambda i: (0, 0, 0)),
        ],
        out_specs=[
            pl.BlockSpec((_BE, HD), lambda i: (i, 0)),
            pl.BlockSpec((_BE, HD), lambda i: (i, 0)),
            pl.BlockSpec((_BE, HD), lambda i: (i, 0)),
        ],
        out_shape=[jax.ShapeDtypeStruct((E, HD), jnp.float32)] * 3,
    )(edge_attr, we_stack, be_stack)


def _bn(y, g, b):
    m = jnp.sum(y, axis=0, keepdims=True) * (1.0 / N)
    d = y - m
    v = jnp.sum(d * d, axis=0, keepdims=True) * (1.0 / N)
    return d * jax.lax.rsqrt(v + 1e-5) * g + b


def _lrelu(y):
    return jnp.where(y >= 0, y, 0.01 * y)


def _mlp_body(h_ref, agg_ref, eps_ref, w1_ref, b1_ref, g1_ref, bb1_ref,
              w2_ref, b2_ref, g2_ref, bb2_ref, out_ref):
    h = h_ref[...]
    z = (1.0 + eps_ref[0, 0]) * h + agg_ref[:N, :] + agg_ref[N:, :]
    y = jnp.dot(z, w1_ref[...], preferred_element_type=jnp.float32,
                precision=_HIGH) + b1_ref[...]
    y = _lrelu(_bn(y, g1_ref[...], bb1_ref[...]))
    y = jnp.dot(y, w2_ref[...], preferred_element_type=jnp.float32,
                precision=_HIGH) + b2_ref[...]
    out_ref[...] = _lrelu(_bn(y, g2_ref[...], bb2_ref[...]))


def _layer_mlp(h, aggp, p):
    args = (h, aggp, p["eps"].reshape(1, 1),
            p["W1"], p["b1"].reshape(1, HD), p["g1"].reshape(1, HD),
            p["bb1"].reshape(1, HD),
            p["W2"], p["b2"].reshape(1, HD), p["g2"].reshape(1, HD),
            p["bb2"].reshape(1, HD))
    return pl.pallas_call(
        _mlp_body,
        out_shape=jax.ShapeDtypeStruct((N, HD), jnp.float32),
    )(*args)


def _final_body(h_ref, batch_ref, wft_ref, wfb_ref, bf_ref, out_ref):
    h = h_ref[...]
    b = batch_ref[...]
    gi = jax.lax.broadcasted_iota(jnp.int32, (N, G), 1)
    oh = (b == gi).astype(jnp.float32)
    pool = jax.lax.dot_general(oh, h, (((0,), (0,)), ((), ())),
                               preferred_element_type=jnp.float32,
                               precision=_HIGH)
    pe = jnp.dot(oh, pool, preferred_element_type=jnp.float32,
                 precision=_HIGH)
    logits = (jnp.dot(h, wft_ref[...], preferred_element_type=jnp.float32,
                      precision=_HIGH)
              + jnp.dot(pe, wfb_ref[...], preferred_element_type=jnp.float32,
                        precision=_HIGH)
              + bf_ref[...])
    m = jnp.max(logits, axis=1, keepdims=True)
    ex = jnp.exp(logits - m)
    out_ref[...] = ex / jnp.sum(ex, axis=1, keepdims=True)


def _final(h, batch2d, wf_top, wf_bot, bf):
    return pl.pallas_call(
        _final_body,
        out_shape=jax.ShapeDtypeStruct((N, C), jnp.float32),
    )(h, batch2d, wf_top, wf_bot, bf.reshape(1, C))


def kernel(x, edge_attr, params, Wf, bf, edge_index, batch):
    src = edge_index[0].astype(jnp.int32)
    dst = edge_index[1].astype(jnp.int32)
    we_stack = jnp.stack([p["We"] for p in params])
    be_stack = jnp.stack([p["be"].reshape(1, HD) for p in params])
    eas = _edge_embed(edge_attr, we_stack, be_stack)

    dst4 = dst.reshape(NW, NSC, SCB, BLK)
    h = x
    for l, p in enumerate(params):
        aggp = _sc_msg(h, eas[l], src, dst4)
        h = _layer_mlp(h, aggp, p)

    return _final(h, batch.astype(jnp.int32).reshape(N, 1),
                  Wf[:HD], Wf[HD:], bf)


# speedup vs baseline: 4.7326x; 1.2202x over previous
import functools

import jax
import jax.numpy as jnp
from jax import lax
from jax.experimental import pallas as pl
from jax.experimental.pallas import tpu as pltpu
from jax.experimental.pallas import tpu_sc as plsc

N = 10000
E = 320000
DE = 16
HD = 128
G = 64
C = 10

NC = 2
NS = 16
NW = NC * NS

EPT = E // NW
BLK = 40
NB = EPT // BLK
SCB = 50
NSC = NB // SCB
ZCH = 40
NCH = N // ZCH
KMAX = (NCH + NS - 1) // NS

def _sc_body(h_hbm, ea_hbm, src_hbm, dst4_hbm, out_hbm,
             sidx_v, didx_v, hrow_v, ea_v, msg_v, agg_sh,
             gsem, esem, ssem):
    c = lax.axis_index("c")
    s = lax.axis_index("s")
    wid = s * NC + c

    def zfill(e, _):
        for cc in range(HD // 16):
            msg_v[0, e, pl.ds(cc * 16, 16)] = jnp.zeros((16,), jnp.float32)
        return 0
    lax.fori_loop(0, BLK, zfill, 0)
    for k in range(KMAX):
        ch = s + k * NS
        @pl.when(ch < NCH)
        def _():
            pltpu.sync_copy(msg_v.at[0], agg_sh.at[pl.ds(ch * ZCH, ZCH), :])
    plsc.subcore_barrier()

    def issue(sc, bl, par):
        base = wid * EPT + sc * SCB * BLK + bl * BLK
        pltpu.async_copy(h_hbm.at[sidx_v.at[pl.ds(bl * BLK, BLK)]],
                         hrow_v.at[par], gsem.at[par])
        pltpu.async_copy(ea_hbm.at[pl.ds(base, BLK), :], ea_v.at[par],
                         esem.at[par])

    def wait_in(par):
        pltpu.make_async_copy(h_hbm.at[pl.ds(0, BLK), :], hrow_v.at[par],
                              gsem.at[par]).wait()
        pltpu.make_async_copy(ea_hbm.at[pl.ds(0, BLK), :], ea_v.at[par],
                              esem.at[par]).wait()

    def drain_scatter(par):
        pltpu.make_async_copy(ea_hbm.at[pl.ds(0, BLK), :], msg_v.at[par],
                              ssem.at[par]).wait()

    def chunk(sc, _):
        pltpu.sync_copy(src_hbm.at[pl.ds(wid * EPT + sc * SCB * BLK,
                                         SCB * BLK)], sidx_v)
        pltpu.sync_copy(dst4_hbm.at[wid, sc], didx_v)
        issue(sc, 0, 0)

        def pair(i, _):
            bl0 = i * 2
            for par in range(2):
                bl = bl0 + par
                nxt = bl + 1
                @pl.when(nxt < SCB)
                def _():
                    issue(sc, nxt, 1 - par)
                wait_in(par)
                @pl.when(bl >= 2)
                def _():
                    drain_scatter(par)

                def row(e, _):
                    for cc in range(HD // 16):
                        sl = pl.ds(cc * 16, 16)
                        msg_v[par, e, sl] = jnp.maximum(
                            hrow_v[par, e, sl] + ea_v[par, e, sl], 0.0)
                    return 0
                lax.fori_loop(0, BLK, row, 0)

                pltpu.async_copy(msg_v.at[par], agg_sh.at[didx_v.at[bl]],
                                 ssem.at[par], add=True)
            return 0
        lax.fori_loop(0, SCB // 2, pair, 0)
        drain_scatter(0)
        drain_scatter(1)
        return 0
    lax.fori_loop(0, NSC, chunk, 0)

    plsc.subcore_barrier()

    for k in range(KMAX):
        ch = s + k * NS
        @pl.when(ch < NCH)
        def _():
            pltpu.sync_copy(agg_sh.at[pl.ds(ch * ZCH, ZCH), :],
                            out_hbm.at[pl.ds(c * N + ch * ZCH, ZCH), :])


_sc_msg = functools.partial(
    pl.kernel,
    out_type=jax.ShapeDtypeStruct((2 * N, HD), jnp.float32),
    mesh=plsc.VectorSubcoreMesh(core_axis_name="c", subcore_axis_name="s"),
    scratch_types=[
        pltpu.VMEM((SCB * BLK,), jnp.int32),
        pltpu.VMEM((SCB, BLK), jnp.int32),
        pltpu.VMEM((2, BLK, HD), jnp.float32),
        pltpu.VMEM((2, BLK, HD), jnp.float32),
        pltpu.VMEM((2, BLK, HD), jnp.float32),
        pltpu.VMEM_SHARED((N, HD), jnp.float32),
        pltpu.SemaphoreType.DMA((2,)),
        pltpu.SemaphoreType.DMA((2,)),
        pltpu.SemaphoreType.DMA((2,)),
    ],
)(_sc_body)


_BE = 4000


def _embed_body(ea_ref, we_ref, be_ref, o_ref):
    o = jnp.dot(ea_ref[...], we_ref[...], preferred_element_type=jnp.float32)
    o_ref[...] = o + be_ref[...]


def _edge_embed(edge_attr, we, be):
    nblk = E // _BE
    return pl.pallas_call(
        _embed_body,
        grid=(nblk,),
        in_specs=[
            pl.BlockSpec((_BE, DE), lambda i: (i, 0)),
            pl.BlockSpec((DE, HD), lambda i: (0, 0)),
            pl.BlockSpec((1, HD), lambda i: (0, 0)),
        ],
        out_specs=pl.BlockSpec((_BE, HD), lambda i: (i, 0)),
        out_shape=jax.ShapeDtypeStruct((E, HD), jnp.float32),
    )(edge_attr, we, be)


def _bn(y, g, b):
    m = jnp.sum(y, axis=0, keepdims=True) * (1.0 / N)
    d = y - m
    v = jnp.sum(d * d, axis=0, keepdims=True) * (1.0 / N)
    return d * jax.lax.rsqrt(v + 1e-5) * g + b


def _lrelu(y):
    return jnp.where(y >= 0, y, 0.01 * y)


def _mlp_body(h_ref, agg_ref, eps_ref, w1_ref, b1_ref, g1_ref, bb1_ref,
              w2_ref, b2_ref, g2_ref, bb2_ref, out_ref):
    h = h_ref[...]
    z = (1.0 + eps_ref[0, 0]) * h + agg_ref[:N, :] + agg_ref[N:, :]
    y = jnp.dot(z, w1_ref[...],
                preferred_element_type=jnp.float32) + b1_ref[...]
    y = _lrelu(_bn(y, g1_ref[...], bb1_ref[...]))
    y = jnp.dot(y, w2_ref[...],
                preferred_element_type=jnp.float32) + b2_ref[...]
    out_ref[...] = _lrelu(_bn(y, g2_ref[...], bb2_ref[...]))


def _layer_mlp(h, aggp, p):
    args = (h, aggp, p["eps"].reshape(1, 1),
            p["W1"], p["b1"].reshape(1, HD), p["g1"].reshape(1, HD),
            p["bb1"].reshape(1, HD),
            p["W2"], p["b2"].reshape(1, HD), p["g2"].reshape(1, HD),
            p["bb2"].reshape(1, HD))
    return pl.pallas_call(
        _mlp_body,
        out_shape=jax.ShapeDtypeStruct((N, HD), jnp.float32),
    )(*args)


def _final_body(h_ref, batch_ref, wft_ref, wfb_ref, bf_ref, out_ref):
    h = h_ref[...]
    b = batch_ref[...]
    gi = jax.lax.broadcasted_iota(jnp.int32, (N, G), 1)
    oh = (b == gi).astype(jnp.float32)
    pool = jax.lax.dot_general(oh, h, (((0,), (0,)), ((), ())),
                               preferred_element_type=jnp.float32)
    pe = jnp.dot(oh, pool, preferred_element_type=jnp.float32)
    logits = (jnp.dot(h, wft_ref[...], preferred_element_type=jnp.float32)
              + jnp.dot(pe, wfb_ref[...], preferred_element_type=jnp.float32)
              + bf_ref[...])
    m = jnp.max(logits, axis=1, keepdims=True)
    ex = jnp.exp(logits - m)
    out_ref[...] = ex / jnp.sum(ex, axis=1, keepdims=True)


def _final(h, batch2d, wf_top, wf_bot, bf):
    return pl.pallas_call(
        _final_body,
        out_shape=jax.ShapeDtypeStruct((N, C), jnp.float32),
    )(h, batch2d, wf_top, wf_bot, bf.reshape(1, C))


def kernel(x, edge_attr, params, Wf, bf, edge_index, batch):
    src = edge_index[0].astype(jnp.int32)
    dst = edge_index[1].astype(jnp.int32)
    eas = [_edge_embed(edge_attr, p["We"], p["be"].reshape(1, HD))
           for p in params]

    dst4 = dst.reshape(NW, NSC, SCB, BLK)
    h = x
    for l, p in enumerate(params):
        aggp = _sc_msg(h, eas[l], src, dst4)
        h = _layer_mlp(h, aggp, p)

    return _final(h, batch.astype(jnp.int32).reshape(N, 1),
                  Wf[:HD], Wf[HD:], bf)


# speedup vs baseline: 5.4019x; 1.1414x over previous
import functools

import jax
import jax.numpy as jnp
from jax import lax
from jax.experimental import pallas as pl
from jax.experimental.pallas import tpu as pltpu
from jax.experimental.pallas import tpu_sc as plsc

N = 10000
E = 320000
DE = 16
HD = 128
G = 64
C = 10

NC = 2
NS = 16
NW = NC * NS

EPT = E // NW
BLK = 40
NB = EPT // BLK
SCB = 50
NSC = NB // SCB
ZCH = 40
NCH = N // ZCH
KMAX = (NCH + NS - 1) // NS

def _sc_body(h_hbm, ea_hbm, src_hbm, dst4_hbm, out_hbm,
             sidx_v, didx_v, hrow_v, ea_v, msg_v, agg_sh,
             gsem, esem, ssem):
    c = lax.axis_index("c")
    s = lax.axis_index("s")
    wid = s * NC + c

    def zfill(e, _):
        for cc in range(HD // 16):
            msg_v[0, e, pl.ds(cc * 16, 16)] = jnp.zeros((16,), jnp.float32)
        return 0
    lax.fori_loop(0, BLK, zfill, 0)
    for k in range(KMAX):
        ch = s + k * NS
        @pl.when(ch < NCH)
        def _():
            pltpu.sync_copy(msg_v.at[0], agg_sh.at[pl.ds(ch * ZCH, ZCH), :])
    plsc.subcore_barrier()

    def issue(sc, bl, par):
        base = wid * EPT + sc * SCB * BLK + bl * BLK
        pltpu.async_copy(h_hbm.at[sidx_v.at[pl.ds(bl * BLK, BLK)]],
                         hrow_v.at[par], gsem.at[par])
        pltpu.async_copy(ea_hbm.at[pl.ds(base, BLK), :], ea_v.at[par],
                         esem.at[par])

    def wait_in(par):
        pltpu.make_async_copy(h_hbm.at[pl.ds(0, BLK), :], hrow_v.at[par],
                              gsem.at[par]).wait()
        pltpu.make_async_copy(ea_hbm.at[pl.ds(0, BLK), :], ea_v.at[par],
                              esem.at[par]).wait()

    def drain_scatter(par):
        pltpu.make_async_copy(ea_hbm.at[pl.ds(0, BLK), :], msg_v.at[par],
                              ssem.at[par]).wait()

    def chunk(sc, _):
        pltpu.sync_copy(src_hbm.at[pl.ds(wid * EPT + sc * SCB * BLK,
                                         SCB * BLK)], sidx_v)
        pltpu.sync_copy(dst4_hbm.at[wid, sc], didx_v)
        issue(sc, 0, 0)

        def pair(i, _):
            bl0 = i * 2
            for par in range(2):
                bl = bl0 + par
                nxt = bl + 1
                @pl.when(nxt < SCB)
                def _():
                    issue(sc, nxt, 1 - par)
                wait_in(par)
                @pl.when(bl >= 2)
                def _():
                    drain_scatter(par)

                def row(e, _):
                    for cc in range(HD // 16):
                        sl = pl.ds(cc * 16, 16)
                        msg_v[par, e, sl] = jnp.maximum(
                            hrow_v[par, e, sl] + ea_v[par, e, sl], 0.0)
                    return 0
                lax.fori_loop(0, BLK, row, 0)

                pltpu.async_copy(msg_v.at[par], agg_sh.at[didx_v.at[bl]],
                                 ssem.at[par], add=True)
            return 0
        lax.fori_loop(0, SCB // 2, pair, 0)
        drain_scatter(0)
        drain_scatter(1)
        return 0
    lax.fori_loop(0, NSC, chunk, 0)

    plsc.subcore_barrier()

    for k in range(KMAX):
        ch = s + k * NS
        @pl.when(ch < NCH)
        def _():
            pltpu.sync_copy(agg_sh.at[pl.ds(ch * ZCH, ZCH), :],
                            out_hbm.at[pl.ds(c * N + ch * ZCH, ZCH), :])


_sc_msg = functools.partial(
    pl.kernel,
    out_type=jax.ShapeDtypeStruct((2 * N, HD), jnp.float32),
    mesh=plsc.VectorSubcoreMesh(core_axis_name="c", subcore_axis_name="s"),
    scratch_types=[
        pltpu.VMEM((SCB * BLK,), jnp.int32),
        pltpu.VMEM((SCB, BLK), jnp.int32),
        pltpu.VMEM((2, BLK, HD), jnp.float32),
        pltpu.VMEM((2, BLK, HD), jnp.float32),
        pltpu.VMEM((2, BLK, HD), jnp.float32),
        pltpu.VMEM_SHARED((N, HD), jnp.float32),
        pltpu.SemaphoreType.DMA((2,)),
        pltpu.SemaphoreType.DMA((2,)),
        pltpu.SemaphoreType.DMA((2,)),
    ],
)(_sc_body)


_BE = 6400


def _embed_body(eat_ref, we_ref, be_ref, o_ref):
    ea = eat_ref[...].T
    o = jnp.dot(ea, we_ref[...], preferred_element_type=jnp.float32,
                precision=jax.lax.Precision.HIGHEST)
    o_ref[...] = o + be_ref[...]


def _edge_embed(edge_attr_t, we, be):
    nblk = E // _BE
    return pl.pallas_call(
        _embed_body,
        grid=(nblk,),
        in_specs=[
            pl.BlockSpec((DE, _BE), lambda i: (0, i)),
            pl.BlockSpec((DE, HD), lambda i: (0, 0)),
            pl.BlockSpec((1, HD), lambda i: (0, 0)),
        ],
        out_specs=pl.BlockSpec((_BE, HD), lambda i: (i, 0)),
        out_shape=jax.ShapeDtypeStruct((E, HD), jnp.float32),
    )(edge_attr_t, we, be)


def _bn(y, g, b):
    m = jnp.sum(y, axis=0, keepdims=True) * (1.0 / N)
    d = y - m
    v = jnp.sum(d * d, axis=0, keepdims=True) * (1.0 / N)
    return d * jax.lax.rsqrt(v + 1e-5) * g + b


def _lrelu(y):
    return jnp.where(y >= 0, y, 0.01 * y)


def _mlp_body(h_ref, agg_ref, eps_ref, w1_ref, b1_ref, g1_ref, bb1_ref,
              w2_ref, b2_ref, g2_ref, bb2_ref, out_ref):
    h = h_ref[...]
    z = (1.0 + eps_ref[0, 0]) * h + agg_ref[:N, :] + agg_ref[N:, :]
    y = jnp.dot(z, w1_ref[...],
                preferred_element_type=jnp.float32) + b1_ref[...]
    y = _lrelu(_bn(y, g1_ref[...], bb1_ref[...]))
    y = jnp.dot(y, w2_ref[...],
                preferred_element_type=jnp.float32) + b2_ref[...]
    out_ref[...] = _lrelu(_bn(y, g2_ref[...], bb2_ref[...]))


def _layer_mlp(h, aggp, p):
    args = (h, aggp, p["eps"].reshape(1, 1),
            p["W1"], p["b1"].reshape(1, HD), p["g1"].reshape(1, HD),
            p["bb1"].reshape(1, HD),
            p["W2"], p["b2"].reshape(1, HD), p["g2"].reshape(1, HD),
            p["bb2"].reshape(1, HD))
    return pl.pallas_call(
        _mlp_body,
        out_shape=jax.ShapeDtypeStruct((N, HD), jnp.float32),
    )(*args)


def _final_body(h_ref, batch_ref, wft_ref, wfb_ref, bf_ref, out_ref):
    h = h_ref[...]
    b = batch_ref[...]
    gi = jax.lax.broadcasted_iota(jnp.int32, (N, G), 1)
    oh = (b == gi).astype(jnp.float32)
    pool = jax.lax.dot_general(oh, h, (((0,), (0,)), ((), ())),
                               preferred_element_type=jnp.float32)
    pe = jnp.dot(oh, pool, preferred_element_type=jnp.float32)
    logits = (jnp.dot(h, wft_ref[...], preferred_element_type=jnp.float32)
              + jnp.dot(pe, wfb_ref[...], preferred_element_type=jnp.float32)
              + bf_ref[...])
    m = jnp.max(logits, axis=1, keepdims=True)
    ex = jnp.exp(logits - m)
    out_ref[...] = ex / jnp.sum(ex, axis=1, keepdims=True)


def _final(h, batch2d, wf_top, wf_bot, bf):
    return pl.pallas_call(
        _final_body,
        out_shape=jax.ShapeDtypeStruct((N, C), jnp.float32),
    )(h, batch2d, wf_top, wf_bot, bf.reshape(1, C))


def kernel(x, edge_attr, params, Wf, bf, edge_index, batch):
    src = edge_index[0].astype(jnp.int32)
    dst = edge_index[1].astype(jnp.int32)
    eas = [_edge_embed(edge_attr.T, p["We"], p["be"].reshape(1, HD))
           for p in params]

    dst4 = dst.reshape(NW, NSC, SCB, BLK)
    h = x
    for l, p in enumerate(params):
        aggp = _sc_msg(h, eas[l], src, dst4)
        h = _layer_mlp(h, aggp, p)

    return _final(h, batch.astype(jnp.int32).reshape(N, 1),
                  Wf[:HD], Wf[HD:], bf)
